# baseline (speedup 1.0000x reference)
import functools

import jax
import jax.numpy as jnp
from jax.experimental import pallas as pl
from jax.experimental.pallas import tpu as pltpu

_N = 8192
_BI = 2048


def _rn_bf16_arr(x):
    u = jax.lax.bitcast_convert_type(x, jnp.uint32)
    u = (u + jnp.uint32(0x7FFF) + ((u >> 16) & jnp.uint32(1))) & jnp.uint32(
        0xFFFF0000
    )
    return jax.lax.bitcast_convert_type(u, jnp.float32)


def _chamfer_body(a_ref, b_ref, out_ref, cmin_ref, aext_ref):
    ni = _N // _BI

    a = a_ref[...]
    b = b_ref[...]
    asq = jnp.sum(a * a, axis=1, keepdims=True)
    bsq = jnp.sum(b * b, axis=1, keepdims=True)
    asq_hi = _rn_bf16_arr(asq)
    asq_lo = asq - asq_hi
    bsq_hi = _rn_bf16_arr(bsq)
    bsq_lo = bsq - bsq_hi
    am2 = _rn_bf16_arr(a) * jnp.float32(-2.0)
    b16 = _rn_bf16_arr(b)
    ones = jnp.ones((_N, 1), jnp.float32)
    zeros = jnp.zeros((_N, 1), jnp.float32)
    aext_ref[...] = jnp.concatenate(
        [am2, asq_hi, asq_lo, ones, ones, zeros], axis=1
    ).astype(jnp.bfloat16)
    b_ext = jnp.concatenate(
        [b16, ones, ones, bsq_hi, bsq_lo, zeros], axis=1
    ).astype(jnp.bfloat16)

    cmin_ref[...] = jnp.full((1, _N), jnp.inf, jnp.float32)

    def i_step(i, row_sum):
        f = jax.lax.dot_general(
            aext_ref[pl.ds(i * _BI, _BI), :],
            b_ext,
            (((1,), (1,)), ((), ())),
            preferred_element_type=jnp.float32,
        )
        cmin_ref[...] = jnp.minimum(
            cmin_ref[...], jnp.min(f, axis=0, keepdims=True)
        )
        rmin = jnp.maximum(jnp.min(f, axis=1, keepdims=True), 0.0)
        return row_sum + jnp.sum(jnp.sqrt(rmin))

    row_sum = jax.lax.fori_loop(0, ni, i_step, jnp.float32(0.0))
    col_sum = jnp.sum(jnp.sqrt(jnp.maximum(cmin_ref[...], 0.0)))
    out_ref[0, 0] = (row_sum + col_sum) / jnp.float32(_N)


@jax.jit
def kernel(pc1, pc2):
    out = pl.pallas_call(
        _chamfer_body,
        out_shape=jax.ShapeDtypeStruct((1, 1), jnp.float32),
        in_specs=[pl.BlockSpec(memory_space=pltpu.VMEM)] * 2,
        out_specs=pl.BlockSpec(memory_space=pltpu.SMEM),
        scratch_shapes=[
            pltpu.VMEM((1, _N), jnp.float32),
            pltpu.VMEM((_N, 8), jnp.bfloat16),
        ],
    )(pc1.reshape(-1, 3), pc2.reshape(-1, 3))
    return out[0, 0]


# speedup vs baseline: 1.1849x; 1.1849x over previous
import functools

import jax
import jax.numpy as jnp
from jax.experimental import pallas as pl
from jax.experimental.pallas import tpu as pltpu

_N = 8192
_BI = 2048


def _chamfer_body(a_ext, b_ext, out_ref, cmin_ref):
    ni = _N // _BI

    cmin_ref[...] = jnp.full((1, _N), jnp.inf, jnp.float32)

    def i_step(i, row_sum):
        f = jax.lax.dot_general(
            a_ext[pl.ds(i * _BI, _BI), :],
            b_ext[...],
            (((1,), (0,)), ((), ())),
            preferred_element_type=jnp.float32,
        )
        cmin_ref[...] = jnp.minimum(
            cmin_ref[...], jnp.min(f, axis=0, keepdims=True)
        )
        rmin = jnp.maximum(jnp.min(f, axis=1, keepdims=True), 0.0)
        return row_sum + jnp.sum(jnp.sqrt(rmin))

    row_sum = jax.lax.fori_loop(0, ni, i_step, jnp.float32(0.0))
    col_sum = jnp.sum(jnp.sqrt(jnp.maximum(cmin_ref[...], 0.0)))
    out_ref[0, 0] = (row_sum + col_sum) / jnp.float32(_N)


def _rn_bf16(x):
    u = jax.lax.bitcast_convert_type(x, jnp.uint32)
    u = (u + jnp.uint32(0x7FFF) + ((u >> 16) & jnp.uint32(1))) & jnp.uint32(
        0xFFFF0000
    )
    return jax.lax.bitcast_convert_type(u, jnp.float32)


@jax.jit
def kernel(pc1, pc2):
    a = pc1.reshape(-1, 3)
    b = pc2.reshape(-1, 3)
    asq = jnp.sum(a * a, axis=1, keepdims=True)
    bsq = jnp.sum(b * b, axis=1, keepdims=True)
    asq_hi = _rn_bf16(asq)
    asq_lo = asq - asq_hi
    bsq_hi = _rn_bf16(bsq)
    bsq_lo = bsq - bsq_hi
    a16 = _rn_bf16(a) * jnp.float32(-2.0)
    b16 = _rn_bf16(b)
    ones = jnp.ones_like(asq)
    zeros = jnp.zeros_like(asq)
    a_ext = jnp.concatenate(
        [a16, asq_hi, asq_lo, ones, ones, zeros], axis=1
    ).astype(jnp.bfloat16)
    b_ext = (
        jnp.concatenate([b16, ones, ones, bsq_hi, bsq_lo, zeros], axis=1)
        .astype(jnp.bfloat16)
        .T
    )
    out = pl.pallas_call(
        _chamfer_body,
        out_shape=jax.ShapeDtypeStruct((1, 1), jnp.float32),
        in_specs=[pl.BlockSpec(memory_space=pltpu.VMEM)] * 2,
        out_specs=pl.BlockSpec(memory_space=pltpu.SMEM),
        scratch_shapes=[pltpu.VMEM((1, _N), jnp.float32)],
    )(a_ext, b_ext)
    return out[0, 0]
